# Initial kernel scaffold; baseline (speedup 1.0000x reference)
import jax
import jax.numpy as jnp
from jax.experimental import pallas as pl


def kernel(x, emb):
    raise NotImplementedError("write your pallas kernel here")


# speedup vs baseline: 2.7617x; 2.7617x over previous
import jax
import jax.numpy as jnp
from jax.experimental import pallas as pl

_DIM = 1024
_SCALE = _DIM ** (-0.5)


def _scale_kernel(emb_ref, o_ref):
    o_ref[...] = emb_ref[...] * _SCALE


def kernel(x, emb):
    seq_len = x.shape[1]
    block = 512
    return pl.pallas_call(
        _scale_kernel,
        grid=(seq_len // block,),
        in_specs=[pl.BlockSpec((block, _DIM), lambda i: (i, 0))],
        out_specs=pl.BlockSpec((block, _DIM), lambda i: (i, 0)),
        out_shape=jax.ShapeDtypeStruct((seq_len, _DIM), emb.dtype),
    )(emb)


# speedup vs baseline: 3.0269x; 1.0960x over previous
import jax
import jax.numpy as jnp
from jax.experimental import pallas as pl

_DIM = 1024
_SCALE = _DIM ** (-0.5)


def _scale_kernel(emb_ref, o_ref):
    o_ref[...] = emb_ref[...] * _SCALE


def kernel(x, emb):
    seq_len = x.shape[1]
    block = 1024
    return pl.pallas_call(
        _scale_kernel,
        grid=(seq_len // block,),
        in_specs=[pl.BlockSpec((block, _DIM), lambda i: (i, 0))],
        out_specs=pl.BlockSpec((block, _DIM), lambda i: (i, 0)),
        out_shape=jax.ShapeDtypeStruct((seq_len, _DIM), emb.dtype),
    )(emb)


# speedup vs baseline: 3.2520x; 1.0744x over previous
import jax
import jax.numpy as jnp
from jax.experimental import pallas as pl

_DIM = 1024
_SCALE = _DIM ** (-0.5)


def _scale_kernel(emb_ref, o_ref):
    o_ref[...] = emb_ref[...] * _SCALE


def kernel(x, emb):
    seq_len = x.shape[1]
    block = 2048
    return pl.pallas_call(
        _scale_kernel,
        grid=(seq_len // block,),
        in_specs=[pl.BlockSpec((block, _DIM), lambda i: (i, 0))],
        out_specs=pl.BlockSpec((block, _DIM), lambda i: (i, 0)),
        out_shape=jax.ShapeDtypeStruct((seq_len, _DIM), emb.dtype),
    )(emb)
